# Initial kernel scaffold; baseline (speedup 1.0000x reference)
import jax
import jax.numpy as jnp
from jax.experimental import pallas as pl


def kernel(x, user_emb, movie_emb, age_emb, occupation_emb, movie_year_emb, rate_year_emb, wide_user, wide_movie, wide_gender, wide_age, wide_occupation, wide_movie_year, wide_rate_year, wide_stat_W, wide_stat_b, wide_cross, W1, b1, W2, b2, W3, b3):
    raise NotImplementedError("write your pallas kernel here")


# speedup vs baseline: 1.1607x; 1.1607x over previous
import functools

import jax
import jax.numpy as jnp
from jax import lax
from jax.experimental import pallas as pl
from jax.experimental.pallas import tpu as pltpu
from jax.experimental.pallas import tpu_sc as plsc

NUM_MY = 82
B = 16384
NW = 32
BPW = B // NW
CH = 128
NCH = BPW // CH
H1 = 256
H2 = 128
R = 2048
F32 = jnp.float32


def _sc_gather_body(*refs):
    tables = refs[0:14]
    idx_hbm = refs[14:22]
    outs = refs[22:36]
    idx_v = refs[36:44]
    dst_v = refs[44:58]
    sem = refs[58]

    wid = lax.axis_index("s") * 2 + lax.axis_index("c")
    base = wid * BPW

    for ih, iv in zip(idx_hbm, idx_v):
        pltpu.sync_copy(ih.at[pl.ds(base, BPW)], iv)

    which = [0, 1, 3, 4, 5, 6, 0, 1, 2, 3, 4, 5, 6, 7]
    copies = []
    for tbl, w, dst in zip(tables, which, dst_v):
        iv = idx_v[w]
        for j in range(NCH):
            copies.append(
                pltpu.async_copy(
                    tbl.at[iv.at[pl.ds(j * CH, CH)]],
                    dst.at[pl.ds(j * CH, CH)],
                    sem,
                ))
    for c in copies:
        c.wait()

    for dst, out in zip(dst_v, outs):
        pltpu.sync_copy(dst, out.at[pl.ds(base, BPW)])


def _sc_gather(tables, idx, interpret=False):
    shapes = [t.shape[1:] for t in tables]
    out_type = [jax.ShapeDtypeStruct((B,) + s, F32) for s in shapes]
    scratch = ([pltpu.VMEM((BPW,), jnp.int32)] * 8
               + [pltpu.VMEM((BPW,) + s, F32) for s in shapes]
               + [pltpu.SemaphoreType.DMA])
    mesh = plsc.VectorSubcoreMesh(core_axis_name="c", subcore_axis_name="s",
                                  num_cores=2)
    fn = pl.kernel(
        _sc_gather_body,
        out_type=out_type,
        mesh=mesh,
        scratch_types=scratch,
        compiler_params=pltpu.CompilerParams(use_tc_tiling_on_sc=False),
        interpret=interpret,
    )
    return fn(*tables, *idx)


def _tc_mlp_body(x_ref, ue, me, ae, oe, mye, rye,
                 wu, wm, wg, wa, wo, wmy, wry, wc,
                 w1u, w1m, w1ae, w1oe, w1my, w1ry, wgs, b1r, w2r, b2r, w3r,
                 scal, out_ref):
    xb = x_ref[...]
    g = xb[:, 2:3]
    s0 = xb[:, 7:8]
    s1 = xb[:, 8:9]
    dot = functools.partial(jnp.dot, preferred_element_type=F32)
    h = (dot(ue[...], w1u[...]) + dot(me[...], w1m[...])
         + dot(ae[...], w1ae[...]) + dot(oe[...], w1oe[...])
         + dot(mye[...], w1my[...]) + dot(rye[...], w1ry[...]))
    h = h + g * wgs[0:1, :] + s0 * wgs[1:2, :] + s1 * wgs[2:3, :] + b1r[...]
    h = jnp.maximum(h, 0.0)
    h2 = jnp.maximum(dot(h, w2r[...]) + b2r[...], 0.0)
    deep = dot(h2, w3r[...]) + scal[0, 3]
    wide = (wu[...] + wm[...] + wg[...] + wa[...] + wo[...] + wmy[...]
            + wry[...] + wc[...]
            + s0 * scal[0, 0] + s1 * scal[0, 1] + scal[0, 2])
    out_ref[...] = jax.nn.sigmoid(0.5 * wide + 0.5 * deep)


def _tc_mlp(x, feats, wides, weights, scal, interpret=False):
    nblk = B // R

    def brow(d):
        return pl.BlockSpec((R, d), lambda i: (i, 0))

    def wfull(a):
        return pl.BlockSpec(a.shape, lambda i: (0, 0))

    in_specs = ([brow(9)]
                + [brow(f.shape[1]) for f in feats]
                + [brow(1) for _ in wides]
                + [wfull(w) for w in weights]
                + [pl.BlockSpec(scal.shape, lambda i: (0, 0),
                                memory_space=pltpu.SMEM)])
    fn = pl.pallas_call(
        _tc_mlp_body,
        grid=(nblk,),
        in_specs=in_specs,
        out_specs=pl.BlockSpec((R, 1), lambda i: (i, 0)),
        out_shape=jax.ShapeDtypeStruct((B, 1), F32),
        compiler_params=pltpu.CompilerParams(
            dimension_semantics=("parallel",)),
        interpret=interpret,
    )
    return fn(x, *feats, *wides, *weights, scal)


def _pad16(t):
    return jnp.pad(t, ((0, 0), (0, 16 - t.shape[1])))


def kernel(x, user_emb, movie_emb, age_emb, occupation_emb, movie_year_emb,
           rate_year_emb, wide_user, wide_movie, wide_gender, wide_age,
           wide_occupation, wide_movie_year, wide_rate_year, wide_stat_W,
           wide_stat_b, wide_cross, W1, b1, W2, b2, W3, b3,
           interpret=False):
    xi = x.astype(jnp.int32)
    uid, mid, gid = xi[:, 0], xi[:, 1], xi[:, 2]
    aid, oid, myid, ryid = xi[:, 3], xi[:, 4], xi[:, 5], xi[:, 6]
    cross = aid * (NUM_MY + 1) + myid
    idx = (uid, mid, gid, aid, oid, myid, ryid, cross)

    tables = (user_emb, movie_emb, _pad16(age_emb), occupation_emb,
              _pad16(movie_year_emb), _pad16(rate_year_emb),
              wide_user[:, 0], wide_movie[:, 0], wide_gender[:, 0],
              wide_age[:, 0], wide_occupation[:, 0], wide_movie_year[:, 0],
              wide_rate_year[:, 0], wide_cross[:, 0])
    g = _sc_gather(tables, idx, interpret=interpret)
    feats = g[0:6]
    wides = [w[:, None] for w in g[6:14]]

    w1t = W1.T
    weights = (
        w1t[0:64], w1t[64:128],
        jnp.pad(w1t[128:136], ((0, 8), (0, 0))),
        w1t[136:152],
        jnp.pad(w1t[152:160], ((0, 8), (0, 0))),
        jnp.pad(w1t[160:164], ((0, 12), (0, 0))),
        w1t[164:167],
        b1.reshape(1, H1), W2.T, b2.reshape(1, H2), W3.T,
    )
    scal = jnp.stack([wide_stat_W[0, 0], wide_stat_W[0, 1],
                      wide_stat_b[0], b3[0]]).reshape(1, 4)
    out = _tc_mlp(x, feats, wides, weights, scal, interpret=interpret)
    return out[:, 0]


# speedup vs baseline: 1.5237x; 1.3128x over previous
import functools

import jax
import jax.numpy as jnp
from jax import lax
from jax.experimental import pallas as pl
from jax.experimental.pallas import tpu as pltpu
from jax.experimental.pallas import tpu_sc as plsc

NUM_MY = 82
B = 16384
NW = 32
BPW = B // NW
CH = 128
NCH = BPW // CH
H1 = 256
H2 = 128
R = 2048
F32 = jnp.float32

WOFF_G, WOFF_A, WOFF_O, WOFF_MY, WOFF_RY, WOFF_X = 0, 2, 10, 32, 115, 126
WVEC = 896


def _sc_gather_body(*refs):
    tables = refs[0:4]
    idx_hbm = refs[4:6]
    outs = refs[6:10]
    idx_v = refs[10:12]
    dst_v = refs[12:16]
    sem = refs[16]

    wid = lax.axis_index("s") * 2 + lax.axis_index("c")
    base = wid * BPW

    for ih, iv in zip(idx_hbm, idx_v):
        pltpu.sync_copy(ih.at[pl.ds(base, BPW)], iv)

    which = [0, 1, 0, 1]
    copies = []
    for tbl, w, dst in zip(tables, which, dst_v):
        iv = idx_v[w]
        for j in range(NCH):
            copies.append(
                pltpu.async_copy(
                    tbl.at[iv.at[pl.ds(j * CH, CH)]],
                    dst.at[pl.ds(j * CH, CH)],
                    sem,
                ))
    for c in copies:
        c.wait()

    for dst, out in zip(dst_v, outs):
        pltpu.sync_copy(dst, out.at[pl.ds(base, BPW)])


def _sc_gather(tables, idx, interpret=False):
    shapes = [t.shape[1:] for t in tables]
    out_type = [jax.ShapeDtypeStruct((B,) + s, F32) for s in shapes]
    scratch = ([pltpu.VMEM((BPW,), jnp.int32)] * 2
               + [pltpu.VMEM((BPW,) + s, F32) for s in shapes]
               + [pltpu.SemaphoreType.DMA])
    mesh = plsc.VectorSubcoreMesh(core_axis_name="c", subcore_axis_name="s",
                                  num_cores=2)
    fn = pl.kernel(
        _sc_gather_body,
        out_type=out_type,
        mesh=mesh,
        scratch_types=scratch,
        compiler_params=pltpu.CompilerParams(use_tc_tiling_on_sc=False),
        interpret=interpret,
    )
    return fn(*tables, *idx)


def _onehot(ids, base, n):
    cols = lax.broadcasted_iota(jnp.int32, (R, n), 1)
    return jnp.where(cols == ids + base, 1.0, 0.0).astype(F32)


def _tc_mlp_body(x_ref, ue, me, wu, wm,
                 w1u, w1m, msm, wgs, b1r, w2r, b2r, w3r, wvec,
                 scal, out_ref):
    xb = x_ref[...]
    xi = xb.astype(jnp.int32)
    gid, aid, oid = xi[:, 2:3], xi[:, 3:4], xi[:, 4:5]
    myid, ryid = xi[:, 5:6], xi[:, 6:7]
    g = xb[:, 2:3]
    s0 = xb[:, 7:8]
    s1 = xb[:, 8:9]
    dot = functools.partial(jnp.dot, preferred_element_type=F32)

    oh = (_onehot(aid, 0, 128) + _onehot(oid, 8, 128)
          + _onehot(myid, 30, 128) + _onehot(ryid, 113, 128))

    h = dot(ue[...], w1u[...]) + dot(me[...], w1m[...]) + dot(oh, msm[...])
    h = h + g * wgs[0:1, :] + s0 * wgs[1:2, :] + s1 * wgs[2:3, :] + b1r[...]
    h = jnp.maximum(h, 0.0)
    h2 = jnp.maximum(dot(h, w2r[...]) + b2r[...], 0.0)
    deep = dot(h2, w3r[...]) + scal[0, 3]

    ohw = (_onehot(gid, WOFF_G, WVEC) + _onehot(aid, WOFF_A, WVEC)
           + _onehot(oid, WOFF_O, WVEC) + _onehot(myid, WOFF_MY, WVEC)
           + _onehot(ryid, WOFF_RY, WVEC)
           + _onehot(aid * (NUM_MY + 1) + myid, WOFF_X, WVEC))
    wide = (wu[...] + wm[...] + dot(ohw, wvec[...])
            + s0 * scal[0, 0] + s1 * scal[0, 1] + scal[0, 2])
    out_ref[...] = jax.nn.sigmoid(0.5 * wide + 0.5 * deep)


def _tc_mlp(x, feats, weights, scal, interpret=False):
    nblk = B // R

    def brow(d):
        return pl.BlockSpec((R, d), lambda i: (i, 0))

    def wfull(a):
        return pl.BlockSpec(a.shape, lambda i: (0, 0))

    in_specs = ([brow(9)]
                + [brow(f.shape[1]) for f in feats]
                + [wfull(w) for w in weights]
                + [pl.BlockSpec(scal.shape, lambda i: (0, 0),
                                memory_space=pltpu.SMEM)])
    fn = pl.pallas_call(
        _tc_mlp_body,
        grid=(nblk,),
        in_specs=in_specs,
        out_specs=pl.BlockSpec((R, 1), lambda i: (i, 0)),
        out_shape=jax.ShapeDtypeStruct((B, 1), F32),
        compiler_params=pltpu.CompilerParams(
            dimension_semantics=("parallel",)),
        interpret=interpret,
    )
    return fn(x, *feats, *weights, scal)


def kernel(x, user_emb, movie_emb, age_emb, occupation_emb, movie_year_emb,
           rate_year_emb, wide_user, wide_movie, wide_gender, wide_age,
           wide_occupation, wide_movie_year, wide_rate_year, wide_stat_W,
           wide_stat_b, wide_cross, W1, b1, W2, b2, W3, b3,
           interpret=False):
    xi = x.astype(jnp.int32)
    uid, mid = xi[:, 0], xi[:, 1]

    tables = (user_emb, movie_emb, wide_user[:, 0], wide_movie[:, 0])
    ue, me, wu, wm = _sc_gather(tables, (uid, mid), interpret=interpret)

    w1t = W1.T
    tsm = jnp.zeros((128, 36), F32)
    tsm = tsm.at[0:8, 0:8].set(age_emb)
    tsm = tsm.at[8:30, 8:24].set(occupation_emb)
    tsm = tsm.at[30:113, 24:32].set(movie_year_emb)
    tsm = tsm.at[113:124, 32:36].set(rate_year_emb)
    msm = tsm @ w1t[128:164]

    wvec = jnp.zeros((WVEC, 1), F32)
    wvec = wvec.at[WOFF_G:WOFF_G + 2].set(wide_gender)
    wvec = wvec.at[WOFF_A:WOFF_A + 8].set(wide_age)
    wvec = wvec.at[WOFF_O:WOFF_O + 22].set(wide_occupation)
    wvec = wvec.at[WOFF_MY:WOFF_MY + 83].set(wide_movie_year)
    wvec = wvec.at[WOFF_RY:WOFF_RY + 11].set(wide_rate_year)
    wvec = wvec.at[WOFF_X:WOFF_X + 664].set(wide_cross)

    weights = (w1t[0:64], w1t[64:128], msm, w1t[164:167],
               b1.reshape(1, H1), W2.T, b2.reshape(1, H2), W3.T, wvec)
    scal = jnp.stack([wide_stat_W[0, 0], wide_stat_W[0, 1],
                      wide_stat_b[0], b3[0]]).reshape(1, 4)
    feats = (ue, me, wu[:, None], wm[:, None])
    out = _tc_mlp(x, feats, weights, scal, interpret=interpret)
    return out[:, 0]


# speedup vs baseline: 1.5286x; 1.0032x over previous
import functools

import jax
import jax.numpy as jnp
from jax import lax
from jax.experimental import pallas as pl
from jax.experimental.pallas import tpu as pltpu
from jax.experimental.pallas import tpu_sc as plsc

NUM_MY = 82
B = 16384
NW = 32
BPW = B // NW
CH = 128
NCH = BPW // CH
H1 = 256
H2 = 128
R = 2048
F32 = jnp.float32

WOFF_G, WOFF_A, WOFF_O, WOFF_MY, WOFF_RY, WOFF_X = 0, 2, 10, 32, 115, 126
WVEC = 896


def _sc_gather_body(*refs):
    tables = refs[0:4]
    idx_hbm = refs[4:6]
    outs = refs[6:10]
    idx_v = refs[10:12]
    dst_v = refs[12:16]
    sem = refs[16]

    wid = lax.axis_index("s") * 2 + lax.axis_index("c")
    base = wid * BPW

    for ih, iv in zip(idx_hbm, idx_v):
        pltpu.sync_copy(ih.at[pl.ds(base, BPW)], iv)

    which = [0, 1, 0, 1]
    copies = []
    for tbl, w, dst in zip(tables, which, dst_v):
        iv = idx_v[w]
        for j in range(NCH):
            copies.append(
                pltpu.async_copy(
                    tbl.at[iv.at[pl.ds(j * CH, CH)]],
                    dst.at[pl.ds(j * CH, CH)],
                    sem,
                ))
    for c in copies:
        c.wait()

    for dst, out in zip(dst_v, outs):
        pltpu.sync_copy(dst, out.at[pl.ds(base, BPW)])


def _sc_gather(tables, idx, interpret=False):
    shapes = [t.shape[1:] for t in tables]
    out_type = [jax.ShapeDtypeStruct((B,) + s, F32) for s in shapes]
    scratch = ([pltpu.VMEM((BPW,), jnp.int32)] * 2
               + [pltpu.VMEM((BPW,) + s, F32) for s in shapes]
               + [pltpu.SemaphoreType.DMA])
    mesh = plsc.VectorSubcoreMesh(core_axis_name="c", subcore_axis_name="s",
                                  num_cores=2)
    fn = pl.kernel(
        _sc_gather_body,
        out_type=out_type,
        mesh=mesh,
        scratch_types=scratch,
        compiler_params=pltpu.CompilerParams(use_tc_tiling_on_sc=False),
        interpret=interpret,
    )
    return fn(*tables, *idx)


def _onehot(ids, base, n):
    cols = lax.broadcasted_iota(jnp.int32, (R, n), 1)
    return jnp.where(cols == ids + base, 1.0, 0.0).astype(F32)


def _tc_mlp_body(x_ref, ue, me, wu, wm,
                 w1u, w1m, msm, wgs, b1r, w2r, b2r, w3r, wvec,
                 scal, out_ref):
    xb = x_ref[...]
    xi = xb.astype(jnp.int32)
    gid, aid, oid = xi[:, 2:3], xi[:, 3:4], xi[:, 4:5]
    myid, ryid = xi[:, 5:6], xi[:, 6:7]
    g = xb[:, 2:3]
    s0 = xb[:, 7:8]
    s1 = xb[:, 8:9]
    dot = functools.partial(jnp.dot, preferred_element_type=F32)

    oh = (_onehot(aid, 0, 128) + _onehot(oid, 8, 128)
          + _onehot(myid, 30, 128) + _onehot(ryid, 113, 128))

    h = dot(ue[...], w1u[...]) + dot(me[...], w1m[...]) + dot(oh, msm[...])
    h = h + g * wgs[0:1, :] + s0 * wgs[1:2, :] + s1 * wgs[2:3, :] + b1r[...]
    h = jnp.maximum(h, 0.0)
    h2 = jnp.maximum(dot(h, w2r[...]) + b2r[...], 0.0)
    deep = dot(h2, w3r[...]) + scal[0, 3]

    ohw = (_onehot(gid, WOFF_G, WVEC) + _onehot(aid, WOFF_A, WVEC)
           + _onehot(oid, WOFF_O, WVEC) + _onehot(myid, WOFF_MY, WVEC)
           + _onehot(ryid, WOFF_RY, WVEC)
           + _onehot(aid * (NUM_MY + 1) + myid, WOFF_X, WVEC))
    wide = (wu[...] + wm[...] + dot(ohw, wvec[...])
            + s0 * scal[0, 0] + s1 * scal[0, 1] + scal[0, 2])
    out_ref[...] = jax.nn.sigmoid(0.5 * wide + 0.5 * deep)


def _tc_mlp(x, feats, weights, scal, interpret=False):
    nblk = B // R

    def brow(d):
        return pl.BlockSpec((R, d), lambda i: (i, 0))

    def wfull(a):
        return pl.BlockSpec(a.shape, lambda i: (0, 0))

    in_specs = ([brow(9)]
                + [brow(f.shape[1]) for f in feats]
                + [wfull(w) for w in weights]
                + [pl.BlockSpec(scal.shape, lambda i: (0, 0),
                                memory_space=pltpu.SMEM)])
    fn = pl.pallas_call(
        _tc_mlp_body,
        grid=(nblk,),
        in_specs=in_specs,
        out_specs=pl.BlockSpec((R, 1), lambda i: (i, 0)),
        out_shape=jax.ShapeDtypeStruct((B, 1), F32),
        compiler_params=pltpu.CompilerParams(
            dimension_semantics=("parallel",)),
        interpret=interpret,
    )
    return fn(x, *feats, *weights, scal)


def kernel(x, user_emb, movie_emb, age_emb, occupation_emb, movie_year_emb,
           rate_year_emb, wide_user, wide_movie, wide_gender, wide_age,
           wide_occupation, wide_movie_year, wide_rate_year, wide_stat_W,
           wide_stat_b, wide_cross, W1, b1, W2, b2, W3, b3,
           interpret=False):
    xi = x.astype(jnp.int32)
    uid, mid = xi[:, 0], xi[:, 1]

    tables = (user_emb, movie_emb, wide_user[:, 0], wide_movie[:, 0])
    ue, me, wu, wm = _sc_gather(tables, (uid, mid), interpret=interpret)
    wu = wu.reshape(B, 1)
    wm = wm.reshape(B, 1)

    w1t = W1.T
    tsm = jnp.zeros((128, 36), F32)
    tsm = tsm.at[0:8, 0:8].set(age_emb)
    tsm = tsm.at[8:30, 8:24].set(occupation_emb)
    tsm = tsm.at[30:113, 24:32].set(movie_year_emb)
    tsm = tsm.at[113:124, 32:36].set(rate_year_emb)
    msm = tsm @ w1t[128:164]

    wvec = jnp.concatenate([
        wide_gender, wide_age, wide_occupation, wide_movie_year,
        wide_rate_year, wide_cross,
        jnp.zeros((WVEC - WOFF_X - 664, 1), F32)])

    weights = (w1t[0:64], w1t[64:128], msm, w1t[164:167],
               b1.reshape(1, H1), W2.T, b2.reshape(1, H2), W3.T, wvec)
    scal = jnp.stack([wide_stat_W[0, 0], wide_stat_W[0, 1],
                      wide_stat_b[0], b3[0]]).reshape(1, 4)
    feats = (ue, me, wu, wm)
    out = _tc_mlp(x, feats, weights, scal, interpret=interpret)
    return out[:, 0]
